# baseline (speedup 1.0000x reference)
import functools

import jax
import jax.numpy as jnp
from jax import lax
from jax.experimental import pallas as pl
from jax.experimental.pallas import tpu as pltpu
from jax.experimental.pallas import tpu_sc as plsc

N = 5308416
NTILES = 32
PER_TILE = N // NTILES
CHUNK = 10368
NCHUNKS = PER_TILE // CHUNK
L = 16

K_MAIN = int(N * 0.05)
K_FALL = max(1, int(N * 0.01))

PASSES = ((21, 11), (10, 11), (0, 10))


def _mesh():
    return plsc.VectorSubcoreMesh(core_axis_name="c", subcore_axis_name="s")


def _wid():
    return lax.axis_index("s") * 2 + lax.axis_index("c")


def _dma_start(src, dst, sem):
    pltpu.make_async_copy(src, dst, sem).start()


def _dma_wait(src, dst, sem):
    pltpu.make_async_copy(src, dst, sem).wait()


ROWS = 6912
ROW_W = 768
ROWS_PER_TILE = ROWS // NTILES
RCHUNK = 24
NRCHUNKS = ROWS_PER_TILE // RCHUNK


def _make_hist2d_kernel(shift, bits, ntargets):
    nb = 1 << bits
    stride = nb + 1
    hsz = ntargets * stride * L
    hi_shift = shift + bits

    def body(*refs):
        if ntargets == 1:
            w_hbm, out_hbm, buf0, buf1, buf2, hist, red, sem0, sem1, sem2 = refs
        else:
            (w_hbm, state_hbm, out_hbm, buf0, buf1, buf2, hist, red, sbuf,
             sem0, sem1, sem2) = refs
        wid = _wid()
        base = wid * ROWS_PER_TILE
        bufs = (buf0, buf1, buf2)
        sems = (sem0, sem1, sem2)

        zeros = jnp.zeros((L,), jnp.int32)
        ones = jnp.ones((L,), jnp.int32)
        iota = jnp.arange(L, dtype=jnp.int32)
        lane_off = iota * stride

        @plsc.parallel_loop(0, hsz // L, unroll=8)
        def _(i):
            hist[pl.ds(i * L, L)] = zeros

        if ntargets == 2:
            pltpu.sync_copy(state_hbm, sbuf)
            p0v = jnp.full((L,), jnp.max(sbuf[0]), jnp.int32)
            p1v = jnp.full((L,), jnp.max(sbuf[3]), jnp.int32)

        def process(buf):
            def row_body(r, _):
                @plsc.parallel_loop(0, ROW_W // L, unroll=8)
                def _(j):
                    v = buf[r, pl.ds(j * L, L)]
                    k = plsc.bitcast(v, jnp.int32) & jnp.int32(0x7FFFFFFF)
                    b = lax.shift_right_logical(k, jnp.int32(shift))
                    if bits < 32 - shift:
                        b = b & jnp.int32(nb - 1)
                    idx = b + lane_off
                    if ntargets == 1:
                        plsc.addupdate_scatter(hist, [idx], ones)
                    else:
                        hi = lax.shift_right_logical(k, jnp.int32(hi_shift))
                        m0 = hi == p0v
                        m1 = hi == p1v
                        idx = idx + jnp.where(m1, jnp.int32(stride * L),
                                              jnp.int32(0))
                        plsc.addupdate_scatter(hist, [idx], ones, mask=m0 | m1)
                return 0
            lax.fori_loop(0, RCHUNK, row_body, 0)

        for b in range(3):
            _dma_start(w_hbm.at[pl.ds(base + b * RCHUNK, RCHUNK)],
                       bufs[b], sems[b])

        def outer(g, _):
            for b in range(3):
                c0 = g * 3 + b
                _dma_wait(w_hbm.at[pl.ds(base + c0 * RCHUNK, RCHUNK)],
                          bufs[b], sems[b])
                process(bufs[b])

                @pl.when(g < NRCHUNKS // 3 - 1)
                def _(b=b, c0=c0):
                    _dma_start(
                        w_hbm.at[pl.ds(base + (c0 + 3) * RCHUNK, RCHUNK)],
                        bufs[b], sems[b])
            return 0
        lax.fori_loop(0, NRCHUNKS // 3, outer, 0)

        for t in range(ntargets):
            @plsc.parallel_loop(0, nb // L, unroll=2)
            def _(c, t=t):
                acc = jnp.zeros((L,), jnp.int32)
                for l in range(L):
                    acc = acc + hist[
                        pl.ds(t * stride * L + l * stride + c * L, L)]
                red[pl.ds(t * nb + c * L, L)] = acc

        for t in range(ntargets):
            pltpu.sync_copy(red.at[pl.ds(t * nb, nb)],
                            out_hbm.at[t * NTILES + wid])

    scratch = [
        pltpu.VMEM((RCHUNK, ROW_W), jnp.float32),
        pltpu.VMEM((RCHUNK, ROW_W), jnp.float32),
        pltpu.VMEM((RCHUNK, ROW_W), jnp.float32),
        pltpu.VMEM((hsz,), jnp.int32),
        pltpu.VMEM((ntargets * nb,), jnp.int32),
    ]
    if ntargets == 2:
        scratch.append(pltpu.VMEM((8, L), jnp.int32))
    scratch += [pltpu.SemaphoreType.DMA] * 3

    return pl.kernel(
        body,
        out_type=jax.ShapeDtypeStruct((ntargets * NTILES, nb), jnp.int32),
        mesh=_mesh(),
        scratch_types=scratch,
        compiler_params=pltpu.CompilerParams(
            needs_layout_passes=False, use_tc_tiling_on_sc=True),
        name=f"sc_hist2d_s{shift}_b{bits}_t{ntargets}",
    )


def _make_hist_kernel(shift, bits, ntargets):
    nb = 1 << bits
    stride = nb + 1
    hsz = ntargets * stride * L
    hi_shift = shift + bits

    def body(*refs):
        if ntargets == 1:
            w_hbm, out_hbm, buf0, buf1, hist, red, sem0, sem1 = refs
            state_v = None
        else:
            w_hbm, state_hbm, out_hbm, buf0, buf1, hist, red, sbuf, sem0, sem1 = refs
            state_v = sbuf
        wid = _wid()
        base = wid * PER_TILE

        zeros = jnp.zeros((L,), jnp.int32)
        ones = jnp.ones((L,), jnp.int32)
        iota = jnp.arange(L, dtype=jnp.int32)
        lane_off = iota * stride

        @plsc.parallel_loop(0, hsz // L, unroll=8)
        def _(i):
            hist[pl.ds(i * L, L)] = zeros

        if ntargets == 2:
            pltpu.sync_copy(state_hbm, sbuf)
            p0 = jnp.max(sbuf[0])
            p1 = jnp.max(sbuf[3])
            p0v = jnp.full((L,), p0, jnp.int32)
            p1v = jnp.full((L,), p1, jnp.int32)

        def process(buf):
            @plsc.parallel_loop(0, CHUNK // L, unroll=8)
            def _(j):
                v = buf[pl.ds(j * L, L)]
                k = plsc.bitcast(v, jnp.int32) & jnp.int32(0x7FFFFFFF)
                b = lax.shift_right_logical(k, jnp.int32(shift))
                if bits < 32 - shift:
                    b = b & jnp.int32(nb - 1)
                idx = b + lane_off
                if ntargets == 1:
                    plsc.addupdate_scatter(hist, [idx], ones)
                else:
                    hi = lax.shift_right_logical(k, jnp.int32(hi_shift))
                    m0 = hi == p0v
                    m1 = hi == p1v
                    idx = idx + jnp.where(m1, jnp.int32(stride * L),
                                          jnp.int32(0))
                    plsc.addupdate_scatter(hist, [idx], ones, mask=m0 | m1)

        _dma_start(w_hbm.at[pl.ds(base, CHUNK)], buf0, sem0)
        _dma_start(w_hbm.at[pl.ds(base + CHUNK, CHUNK)], buf1, sem1)

        def outer(g, _):
            off = base + g * (2 * CHUNK)
            _dma_wait(w_hbm.at[pl.ds(off, CHUNK)], buf0, sem0)
            process(buf0)

            @pl.when(g < NCHUNKS // 2 - 1)
            def _():
                _dma_start(w_hbm.at[pl.ds(off + 2 * CHUNK, CHUNK)], buf0, sem0)

            _dma_wait(w_hbm.at[pl.ds(off + CHUNK, CHUNK)], buf1, sem1)
            process(buf1)

            @pl.when(g < NCHUNKS // 2 - 1)
            def _():
                _dma_start(w_hbm.at[pl.ds(off + 3 * CHUNK, CHUNK)], buf1, sem1)
            return 0
        lax.fori_loop(0, NCHUNKS // 2, outer, 0)

        for t in range(ntargets):
            @plsc.parallel_loop(0, nb // L, unroll=2)
            def _(c, t=t):
                acc = jnp.zeros((L,), jnp.int32)
                for l in range(L):
                    acc = acc + hist[
                        pl.ds(t * stride * L + l * stride + c * L, L)]
                red[pl.ds(t * nb + c * L, L)] = acc

        for t in range(ntargets):
            pltpu.sync_copy(red.at[pl.ds(t * nb, nb)],
                            out_hbm.at[t * NTILES + wid])

    scratch = [
        pltpu.VMEM((CHUNK,), jnp.float32),
        pltpu.VMEM((CHUNK,), jnp.float32),
        pltpu.VMEM((hsz,), jnp.int32),
        pltpu.VMEM((ntargets * nb,), jnp.int32),
    ]
    if ntargets == 2:
        scratch.append(pltpu.VMEM((8, L), jnp.int32))
    scratch += [pltpu.SemaphoreType.DMA, pltpu.SemaphoreType.DMA]

    return pl.kernel(
        body,
        out_type=jax.ShapeDtypeStruct((ntargets * NTILES, nb), jnp.int32),
        mesh=_mesh(),
        scratch_types=scratch,
        compiler_params=pltpu.CompilerParams(needs_layout_passes=False),
        name=f"sc_hist_s{shift}_b{bits}_t{ntargets}",
    )


def _make_walk_kernel(bits, shared_hist):
    nb = 1 << bits

    def body(hists_hbm, state_hbm, out_state_hbm, tbuf, ghist, sbuf, obuf):
        wid = _wid()
        for t in range(2):
            @pl.when(wid == t)
            def _(t=t):
                pltpu.sync_copy(state_hbm, sbuf)
                prefix = jnp.max(sbuf[3 * t + 0])
                rank = jnp.max(sbuf[3 * t + 1])
                bacc = jnp.max(sbuf[3 * t + 2])

                if shared_hist:
                    rows0 = 0
                elif t == 1:
                    rows0 = NTILES
                else:
                    p_other = jnp.max(sbuf[3])
                    rows0 = jnp.where(prefix == p_other, NTILES, 0)
                pltpu.sync_copy(hists_hbm.at[pl.ds(rows0, NTILES)], tbuf)

                @plsc.parallel_loop(0, nb // L, unroll=4)
                def _(c):
                    acc = jnp.zeros((L,), jnp.int32)
                    for r in range(NTILES):
                        acc = acc + tbuf[r, pl.ds(c * L, L)]
                    ghist[pl.ds(c * L, L)] = acc

                def walk_body(c, carry):
                    tot, binv, belowv = carry
                    h = ghist[pl.ds(c * L, L)]
                    cum = plsc.cumsum(h) + tot
                    m = cum <= rank
                    binv = binv + jnp.where(m, jnp.int32(1), jnp.int32(0))
                    belowv = belowv + jnp.where(m, h, jnp.int32(0))
                    return (tot + jnp.sum(h), binv, belowv)

                z = jnp.zeros((L,), jnp.int32)
                _, binv, belowv = lax.fori_loop(
                    0, nb // L, walk_body, (jnp.int32(0), z, z))
                binidx = jnp.sum(binv)
                below = jnp.sum(belowv)

                obuf[0] = jnp.full(
                    (L,), lax.shift_left(prefix, jnp.int32(bits)) | binidx,
                    jnp.int32)
                obuf[1] = jnp.full((L,), rank - below, jnp.int32)
                obuf[2] = jnp.full((L,), bacc + below, jnp.int32)
                pltpu.sync_copy(obuf.at[pl.ds(0, 3)],
                                out_state_hbm.at[pl.ds(3 * t, 3)])

    return pl.kernel(
        body,
        out_type=jax.ShapeDtypeStruct((8, L), jnp.int32),
        mesh=_mesh(),
        scratch_types=[
            pltpu.VMEM((NTILES, nb), jnp.int32),
            pltpu.VMEM((nb,), jnp.int32),
            pltpu.VMEM((8, L), jnp.int32),
            pltpu.VMEM((3, L), jnp.int32),
        ],
        compiler_params=pltpu.CompilerParams(needs_layout_passes=False),
        name=f"sc_walk_b{bits}",
    )


def _mask_body(state_ref, w_ref, o_ref, ratio_ref):
    key_fall = state_ref[0, 0]
    key_main = state_ref[3, 0]
    below_main = state_ref[5, 0]
    use_fall = below_main == jnp.int32(0)
    thr_key = jnp.where(use_fall, key_fall, key_main)
    thr = lax.bitcast_convert_type(thr_key, jnp.float32)
    w = w_ref[...]
    o_ref[...] = w * (jnp.abs(w) >= thr).astype(jnp.float32)

    @pl.when(pl.program_id(0) == 0)
    def _():
        count = jnp.where(use_fall, jnp.int32(K_FALL), below_main)
        ratio_ref[0, 0] = jnp.clip(
            count.astype(jnp.float32) / jnp.float32(N), 0.01, 0.05)


def _mask_tc(w2d, state):
    rows, cols = w2d.shape
    block = rows // 8
    return pl.pallas_call(
        _mask_body,
        grid=(8,),
        in_specs=[
            pl.BlockSpec(memory_space=pltpu.SMEM),
            pl.BlockSpec((block, cols), lambda i: (i, 0)),
        ],
        out_specs=[
            pl.BlockSpec((block, cols), lambda i: (i, 0)),
            pl.BlockSpec(memory_space=pltpu.SMEM),
        ],
        out_shape=[
            jax.ShapeDtypeStruct((rows, cols), jnp.float32),
            jax.ShapeDtypeStruct((1, 1), jnp.float32),
        ],
    )(state, w2d)


def kernel(weight):
    wp = jnp.transpose(weight, (2, 3, 0, 1)).reshape(6912, 768)

    state0 = jnp.tile(
        jnp.array([0, K_FALL, 0, 0, K_MAIN, 0, 0, 0],
                  jnp.int32)[:, None], (1, L))

    hist1 = _make_hist2d_kernel(*PASSES[0], ntargets=1)
    hist2 = _make_hist2d_kernel(*PASSES[1], ntargets=2)
    hist3 = _make_hist2d_kernel(*PASSES[2], ntargets=2)
    walk1 = _make_walk_kernel(PASSES[0][1], shared_hist=True)
    walk2 = _make_walk_kernel(PASSES[1][1], shared_hist=False)
    walk3 = _make_walk_kernel(PASSES[2][1], shared_hist=False)

    h1 = hist1(wp)
    s1 = walk1(h1, state0)
    h2 = hist2(wp, s1)
    s2 = walk2(h2, s1)
    h3 = hist3(wp, s2)
    s3 = walk3(h3, s2)

    masked_p, ratio = _mask_tc(wp, s3)
    masked = jnp.transpose(masked_p.reshape(3, 3, 768, 768), (2, 3, 0, 1))
    return masked, ratio[0, 0]


# speedup vs baseline: 1.0353x; 1.0353x over previous
import functools

import jax
import jax.numpy as jnp
from jax import lax
from jax.experimental import pallas as pl
from jax.experimental.pallas import tpu as pltpu
from jax.experimental.pallas import tpu_sc as plsc

N = 5308416
NTILES = 32
PER_TILE = N // NTILES
CHUNK = 10368
NCHUNKS = PER_TILE // CHUNK
L = 16

K_MAIN = int(N * 0.05)
K_FALL = max(1, int(N * 0.01))

PASSES = ((21, 11), (10, 11), (0, 10))


def _mesh():
    return plsc.VectorSubcoreMesh(core_axis_name="c", subcore_axis_name="s")


def _wid():
    return lax.axis_index("s") * 2 + lax.axis_index("c")


def _dma_start(src, dst, sem):
    pltpu.make_async_copy(src, dst, sem).start()


def _dma_wait(src, dst, sem):
    pltpu.make_async_copy(src, dst, sem).wait()


ROWS = 6912
ROW_W = 768
ROWS_PER_TILE = ROWS // NTILES
RCHUNK = 24
NRCHUNKS = ROWS_PER_TILE // RCHUNK


def _make_hist2d_kernel(shift, bits, ntargets):
    nb = 1 << bits
    stride = nb + 1
    hsz = ntargets * stride * L
    hi_shift = shift + bits

    def body(*refs):
        if ntargets == 1:
            w_hbm, out_hbm, buf0, buf1, buf2, hist, red, sem0, sem1, sem2 = refs
        else:
            (w_hbm, state_hbm, out_hbm, buf0, buf1, buf2, hist, red, sbuf,
             sem0, sem1, sem2) = refs
        wid = _wid()
        base = wid * ROWS_PER_TILE
        bufs = (buf0, buf1, buf2)
        sems = (sem0, sem1, sem2)

        zeros = jnp.zeros((L,), jnp.int32)
        ones = jnp.ones((L,), jnp.int32)
        iota = jnp.arange(L, dtype=jnp.int32)
        lane_off = iota * stride

        @plsc.parallel_loop(0, hsz // L, unroll=8)
        def _(i):
            hist[pl.ds(i * L, L)] = zeros

        if ntargets == 2:
            pltpu.sync_copy(state_hbm, sbuf)
            p0v = jnp.full((L,), jnp.max(sbuf[0]), jnp.int32)
            p1v = jnp.full((L,), jnp.max(sbuf[3]), jnp.int32)

        vregs_per_row = ROW_W // L

        def process(buf):
            @plsc.parallel_loop(0, RCHUNK * vregs_per_row, unroll=8)
            def _(q):
                r = q // vregs_per_row
                j = q - r * vregs_per_row
                v = buf[r, pl.ds(j * L, L)]
                k = plsc.bitcast(v, jnp.int32) & jnp.int32(0x7FFFFFFF)
                b = lax.shift_right_logical(k, jnp.int32(shift))
                if bits < 32 - shift:
                    b = b & jnp.int32(nb - 1)
                idx = b + lane_off
                if ntargets == 1:
                    plsc.addupdate_scatter(hist, [idx], ones)
                else:
                    hi = lax.shift_right_logical(k, jnp.int32(hi_shift))
                    m0 = hi == p0v
                    m1 = hi == p1v
                    idx = idx + jnp.where(m1, jnp.int32(stride * L),
                                          jnp.int32(0))
                    plsc.addupdate_scatter(hist, [idx], ones, mask=m0 | m1)

        for b in range(3):
            _dma_start(w_hbm.at[pl.ds(base + b * RCHUNK, RCHUNK)],
                       bufs[b], sems[b])

        def outer(g, _):
            for b in range(3):
                c0 = g * 3 + b
                _dma_wait(w_hbm.at[pl.ds(base + c0 * RCHUNK, RCHUNK)],
                          bufs[b], sems[b])
                process(bufs[b])

                @pl.when(g < NRCHUNKS // 3 - 1)
                def _(b=b, c0=c0):
                    _dma_start(
                        w_hbm.at[pl.ds(base + (c0 + 3) * RCHUNK, RCHUNK)],
                        bufs[b], sems[b])
            return 0
        lax.fori_loop(0, NRCHUNKS // 3, outer, 0)

        for t in range(ntargets):
            @plsc.parallel_loop(0, nb // L, unroll=2)
            def _(c, t=t):
                acc = jnp.zeros((L,), jnp.int32)
                for l in range(L):
                    acc = acc + hist[
                        pl.ds(t * stride * L + l * stride + c * L, L)]
                red[pl.ds(t * nb + c * L, L)] = acc

        for t in range(ntargets):
            pltpu.sync_copy(red.at[pl.ds(t * nb, nb)],
                            out_hbm.at[t * NTILES + wid])

    scratch = [
        pltpu.VMEM((RCHUNK, ROW_W), jnp.float32),
        pltpu.VMEM((RCHUNK, ROW_W), jnp.float32),
        pltpu.VMEM((RCHUNK, ROW_W), jnp.float32),
        pltpu.VMEM((hsz,), jnp.int32),
        pltpu.VMEM((ntargets * nb,), jnp.int32),
    ]
    if ntargets == 2:
        scratch.append(pltpu.VMEM((8, L), jnp.int32))
    scratch += [pltpu.SemaphoreType.DMA] * 3

    return pl.kernel(
        body,
        out_type=jax.ShapeDtypeStruct((ntargets * NTILES, nb), jnp.int32),
        mesh=_mesh(),
        scratch_types=scratch,
        compiler_params=pltpu.CompilerParams(
            needs_layout_passes=False, use_tc_tiling_on_sc=True),
        name=f"sc_hist2d_s{shift}_b{bits}_t{ntargets}",
    )


def _make_hist_kernel(shift, bits, ntargets):
    nb = 1 << bits
    stride = nb + 1
    hsz = ntargets * stride * L
    hi_shift = shift + bits

    def body(*refs):
        if ntargets == 1:
            w_hbm, out_hbm, buf0, buf1, hist, red, sem0, sem1 = refs
            state_v = None
        else:
            w_hbm, state_hbm, out_hbm, buf0, buf1, hist, red, sbuf, sem0, sem1 = refs
            state_v = sbuf
        wid = _wid()
        base = wid * PER_TILE

        zeros = jnp.zeros((L,), jnp.int32)
        ones = jnp.ones((L,), jnp.int32)
        iota = jnp.arange(L, dtype=jnp.int32)
        lane_off = iota * stride

        @plsc.parallel_loop(0, hsz // L, unroll=8)
        def _(i):
            hist[pl.ds(i * L, L)] = zeros

        if ntargets == 2:
            pltpu.sync_copy(state_hbm, sbuf)
            p0 = jnp.max(sbuf[0])
            p1 = jnp.max(sbuf[3])
            p0v = jnp.full((L,), p0, jnp.int32)
            p1v = jnp.full((L,), p1, jnp.int32)

        def process(buf):
            @plsc.parallel_loop(0, CHUNK // L, unroll=8)
            def _(j):
                v = buf[pl.ds(j * L, L)]
                k = plsc.bitcast(v, jnp.int32) & jnp.int32(0x7FFFFFFF)
                b = lax.shift_right_logical(k, jnp.int32(shift))
                if bits < 32 - shift:
                    b = b & jnp.int32(nb - 1)
                idx = b + lane_off
                if ntargets == 1:
                    plsc.addupdate_scatter(hist, [idx], ones)
                else:
                    hi = lax.shift_right_logical(k, jnp.int32(hi_shift))
                    m0 = hi == p0v
                    m1 = hi == p1v
                    idx = idx + jnp.where(m1, jnp.int32(stride * L),
                                          jnp.int32(0))
                    plsc.addupdate_scatter(hist, [idx], ones, mask=m0 | m1)

        _dma_start(w_hbm.at[pl.ds(base, CHUNK)], buf0, sem0)
        _dma_start(w_hbm.at[pl.ds(base + CHUNK, CHUNK)], buf1, sem1)

        def outer(g, _):
            off = base + g * (2 * CHUNK)
            _dma_wait(w_hbm.at[pl.ds(off, CHUNK)], buf0, sem0)
            process(buf0)

            @pl.when(g < NCHUNKS // 2 - 1)
            def _():
                _dma_start(w_hbm.at[pl.ds(off + 2 * CHUNK, CHUNK)], buf0, sem0)

            _dma_wait(w_hbm.at[pl.ds(off + CHUNK, CHUNK)], buf1, sem1)
            process(buf1)

            @pl.when(g < NCHUNKS // 2 - 1)
            def _():
                _dma_start(w_hbm.at[pl.ds(off + 3 * CHUNK, CHUNK)], buf1, sem1)
            return 0
        lax.fori_loop(0, NCHUNKS // 2, outer, 0)

        for t in range(ntargets):
            @plsc.parallel_loop(0, nb // L, unroll=2)
            def _(c, t=t):
                acc = jnp.zeros((L,), jnp.int32)
                for l in range(L):
                    acc = acc + hist[
                        pl.ds(t * stride * L + l * stride + c * L, L)]
                red[pl.ds(t * nb + c * L, L)] = acc

        for t in range(ntargets):
            pltpu.sync_copy(red.at[pl.ds(t * nb, nb)],
                            out_hbm.at[t * NTILES + wid])

    scratch = [
        pltpu.VMEM((CHUNK,), jnp.float32),
        pltpu.VMEM((CHUNK,), jnp.float32),
        pltpu.VMEM((hsz,), jnp.int32),
        pltpu.VMEM((ntargets * nb,), jnp.int32),
    ]
    if ntargets == 2:
        scratch.append(pltpu.VMEM((8, L), jnp.int32))
    scratch += [pltpu.SemaphoreType.DMA, pltpu.SemaphoreType.DMA]

    return pl.kernel(
        body,
        out_type=jax.ShapeDtypeStruct((ntargets * NTILES, nb), jnp.int32),
        mesh=_mesh(),
        scratch_types=scratch,
        compiler_params=pltpu.CompilerParams(needs_layout_passes=False),
        name=f"sc_hist_s{shift}_b{bits}_t{ntargets}",
    )


def _make_walk_kernel(bits, shared_hist):
    nb = 1 << bits

    def body(hists_hbm, state_hbm, out_state_hbm, tbuf, ghist, sbuf, obuf):
        wid = _wid()
        for t in range(2):
            @pl.when(wid == t)
            def _(t=t):
                pltpu.sync_copy(state_hbm, sbuf)
                prefix = jnp.max(sbuf[3 * t + 0])
                rank = jnp.max(sbuf[3 * t + 1])
                bacc = jnp.max(sbuf[3 * t + 2])

                if shared_hist:
                    rows0 = 0
                elif t == 1:
                    rows0 = NTILES
                else:
                    p_other = jnp.max(sbuf[3])
                    rows0 = jnp.where(prefix == p_other, NTILES, 0)
                pltpu.sync_copy(hists_hbm.at[pl.ds(rows0, NTILES)], tbuf)

                @plsc.parallel_loop(0, nb // L, unroll=4)
                def _(c):
                    acc = jnp.zeros((L,), jnp.int32)
                    for r in range(NTILES):
                        acc = acc + tbuf[r, pl.ds(c * L, L)]
                    ghist[pl.ds(c * L, L)] = acc

                def walk_body(c, carry):
                    tot, binv, belowv = carry
                    h = ghist[pl.ds(c * L, L)]
                    cum = plsc.cumsum(h) + tot
                    m = cum <= rank
                    binv = binv + jnp.where(m, jnp.int32(1), jnp.int32(0))
                    belowv = belowv + jnp.where(m, h, jnp.int32(0))
                    return (tot + jnp.sum(h), binv, belowv)

                z = jnp.zeros((L,), jnp.int32)
                _, binv, belowv = lax.fori_loop(
                    0, nb // L, walk_body, (jnp.int32(0), z, z))
                binidx = jnp.sum(binv)
                below = jnp.sum(belowv)

                obuf[0] = jnp.full(
                    (L,), lax.shift_left(prefix, jnp.int32(bits)) | binidx,
                    jnp.int32)
                obuf[1] = jnp.full((L,), rank - below, jnp.int32)
                obuf[2] = jnp.full((L,), bacc + below, jnp.int32)
                pltpu.sync_copy(obuf.at[pl.ds(0, 3)],
                                out_state_hbm.at[pl.ds(3 * t, 3)])

    return pl.kernel(
        body,
        out_type=jax.ShapeDtypeStruct((8, L), jnp.int32),
        mesh=_mesh(),
        scratch_types=[
            pltpu.VMEM((NTILES, nb), jnp.int32),
            pltpu.VMEM((nb,), jnp.int32),
            pltpu.VMEM((8, L), jnp.int32),
            pltpu.VMEM((3, L), jnp.int32),
        ],
        compiler_params=pltpu.CompilerParams(needs_layout_passes=False),
        name=f"sc_walk_b{bits}",
    )


def _mask_body(state_ref, w_ref, o_ref, ratio_ref):
    key_fall = state_ref[0, 0]
    key_main = state_ref[3, 0]
    below_main = state_ref[5, 0]
    use_fall = below_main == jnp.int32(0)
    thr_key = jnp.where(use_fall, key_fall, key_main)
    thr = lax.bitcast_convert_type(thr_key, jnp.float32)
    w = w_ref[...]
    o_ref[...] = w * (jnp.abs(w) >= thr).astype(jnp.float32)

    @pl.when(pl.program_id(0) == 0)
    def _():
        count = jnp.where(use_fall, jnp.int32(K_FALL), below_main)
        ratio_ref[0, 0] = jnp.clip(
            count.astype(jnp.float32) / jnp.float32(N), 0.01, 0.05)


def _mask_tc(w2d, state):
    rows, cols = w2d.shape
    block = rows // 8
    return pl.pallas_call(
        _mask_body,
        grid=(8,),
        in_specs=[
            pl.BlockSpec(memory_space=pltpu.SMEM),
            pl.BlockSpec((block, cols), lambda i: (i, 0)),
        ],
        out_specs=[
            pl.BlockSpec((block, cols), lambda i: (i, 0)),
            pl.BlockSpec(memory_space=pltpu.SMEM),
        ],
        out_shape=[
            jax.ShapeDtypeStruct((rows, cols), jnp.float32),
            jax.ShapeDtypeStruct((1, 1), jnp.float32),
        ],
    )(state, w2d)


def kernel(weight):
    wp = jnp.transpose(weight, (2, 3, 0, 1)).reshape(6912, 768)

    state0 = jnp.tile(
        jnp.array([0, K_FALL, 0, 0, K_MAIN, 0, 0, 0],
                  jnp.int32)[:, None], (1, L))

    hist1 = _make_hist2d_kernel(*PASSES[0], ntargets=1)
    hist2 = _make_hist2d_kernel(*PASSES[1], ntargets=2)
    hist3 = _make_hist2d_kernel(*PASSES[2], ntargets=2)
    walk1 = _make_walk_kernel(PASSES[0][1], shared_hist=True)
    walk2 = _make_walk_kernel(PASSES[1][1], shared_hist=False)
    walk3 = _make_walk_kernel(PASSES[2][1], shared_hist=False)

    h1 = hist1(wp)
    s1 = walk1(h1, state0)
    h2 = hist2(wp, s1)
    s2 = walk2(h2, s1)
    h3 = hist3(wp, s2)
    s3 = walk3(h3, s2)

    masked_p, ratio = _mask_tc(wp, s3)
    masked = jnp.transpose(masked_p.reshape(3, 3, 768, 768), (2, 3, 0, 1))
    return masked, ratio[0, 0]
